# Initial kernel scaffold; baseline (speedup 1.0000x reference)
import jax
import jax.numpy as jnp
from jax.experimental import pallas as pl


def kernel(x, edge_index, fake_x, fake_edge_index, treat_idx, control_idx, W1, as1, ad1, b1, W2, as2, ad2, b2, Ws, bs, Wy1, by1, Wy0, by0, Wp, bp):
    raise NotImplementedError("write your pallas kernel here")


# speedup vs baseline: 16.0150x; 16.0150x over previous
import functools
import jax
import jax.numpy as jnp
from jax import lax
from jax.experimental import pallas as pl
from jax.experimental.pallas import tpu as pltpu
from jax.experimental.pallas import tpu_sc as plsc

NC = 2
NS = 16
LANES = 16
NW = NC * NS

BN = 400
EPAD = 327680
CHUNK = 64
NCHUNKS = EPAD // NW // CHUNK
CBATCH = 32
NPAD = 10240
TPAD = 5120


def _leaky(x, slope):
    return jnp.maximum(x, slope * x)


def _dense_attn(X, W, av_s, av_d):
    G, n, d = X.shape

    def body(x_ref, w_ref, avs_ref, avd_ref, h_ref, als_ref, ald_ref, exs_ref):
        x = x_ref[0]
        h = jnp.dot(x, w_ref[...], preferred_element_type=jnp.float32)
        als = jnp.sum(h * avs_ref[...], axis=1, keepdims=True)
        ald = jnp.sum(h * avd_ref[...], axis=1, keepdims=True)
        a = als + ald
        h_ref[0] = h
        als_ref[0] = als
        ald_ref[0] = ald
        exs_ref[0] = jnp.exp(_leaky(a, 0.2))

    return pl.pallas_call(
        body,
        grid=(G, n // BN),
        in_specs=[
            pl.BlockSpec((1, BN, d), lambda g, i: (g, i, 0)),
            pl.BlockSpec((d, d), lambda g, i: (0, 0)),
            pl.BlockSpec((1, d), lambda g, i: (0, 0)),
            pl.BlockSpec((1, d), lambda g, i: (0, 0)),
        ],
        out_specs=[
            pl.BlockSpec((1, BN, d), lambda g, i: (g, i, 0)),
            pl.BlockSpec((1, BN, 1), lambda g, i: (g, i, 0)),
            pl.BlockSpec((1, BN, 1), lambda g, i: (g, i, 0)),
            pl.BlockSpec((1, BN, 1), lambda g, i: (g, i, 0)),
        ],
        out_shape=[
            jax.ShapeDtypeStruct((G, n, d), jnp.float32),
            jax.ShapeDtypeStruct((G, n, 1), jnp.float32),
            jax.ShapeDtypeStruct((G, n, 1), jnp.float32),
            jax.ShapeDtypeStruct((G, n, 1), jnp.float32),
        ],
    )(X, W, av_s, av_d)


def _combine1(PW, PD, Hm, EXS, b):
    G, n, d = Hm.shape

    def body(pw_ref, pd_ref, h_ref, exs_ref, b_ref, z_ref):
        pw = pw_ref[0]
        pd = pd_ref[0]
        h = h_ref[0]
        exs = exs_ref[0]
        num = pw[0] + pw[1] + exs * h
        den = jnp.sum(pd, axis=0) + exs
        z = num / (den + 1e-16) + b_ref[...]
        z_ref[0] = jnp.maximum(z, 0.0)

    return pl.pallas_call(
        body,
        grid=(G, n // BN),
        in_specs=[
            pl.BlockSpec((1, NC, BN, d), lambda g, i: (g, 0, i, 0)),
            pl.BlockSpec((1, NW, BN, 1), lambda g, i: (g, 0, i, 0)),
            pl.BlockSpec((1, BN, d), lambda g, i: (g, i, 0)),
            pl.BlockSpec((1, BN, 1), lambda g, i: (g, i, 0)),
            pl.BlockSpec((1, d), lambda g, i: (0, 0)),
        ],
        out_specs=pl.BlockSpec((1, BN, d), lambda g, i: (g, i, 0)),
        out_shape=jax.ShapeDtypeStruct((G, n, d), jnp.float32),
    )(PW, PD, Hm, EXS, b)


def _combine2_heads(PW, PD, Hm, EXS, b2, Ws, bs, Wy1, by1, Wy0, by0, Wp, bp):
    G, n, d = Hm.shape

    def body(pw_ref, pd_ref, h_ref, exs_ref, b2_ref, ws_ref, bs_ref,
             wy1_ref, by1_ref, wy0_ref, by0_ref, wp_ref, bp_ref,
             z_ref, u1_ref, u0_ref, tp_ref):
        pw = pw_ref[0]
        pd = pd_ref[0]
        h = h_ref[0]
        exs = exs_ref[0]
        num = pw[0] + pw[1] + exs * h
        den = jnp.sum(pd, axis=0) + exs
        z = num / (den + 1e-16) + b2_ref[...]
        z_ref[0] = z
        hS = _leaky(jnp.dot(z, ws_ref[...], preferred_element_type=jnp.float32)
                    + bs_ref[...], 0.01)
        u1_ref[0] = _leaky(jnp.dot(hS, wy1_ref[...],
                                   preferred_element_type=jnp.float32)
                           + by1_ref[...], 0.01)
        u0_ref[0] = _leaky(jnp.dot(hS, wy0_ref[...],
                                   preferred_element_type=jnp.float32)
                           + by0_ref[...], 0.01)
        tp_ref[0] = _leaky(jnp.dot(z, wp_ref[...],
                                   preferred_element_type=jnp.float32)
                           + bp_ref[...], 0.01)

    return pl.pallas_call(
        body,
        grid=(G, n // BN),
        in_specs=[
            pl.BlockSpec((1, NC, BN, d), lambda g, i: (g, 0, i, 0)),
            pl.BlockSpec((1, NW, BN, 1), lambda g, i: (g, 0, i, 0)),
            pl.BlockSpec((1, BN, d), lambda g, i: (g, i, 0)),
            pl.BlockSpec((1, BN, 1), lambda g, i: (g, i, 0)),
            pl.BlockSpec((1, d), lambda g, i: (0, 0)),
            pl.BlockSpec((d, d), lambda g, i: (0, 0)),
            pl.BlockSpec((1, d), lambda g, i: (0, 0)),
            pl.BlockSpec((d, 1), lambda g, i: (0, 0)),
            pl.BlockSpec((1, 1), lambda g, i: (0, 0)),
            pl.BlockSpec((d, 1), lambda g, i: (0, 0)),
            pl.BlockSpec((1, 1), lambda g, i: (0, 0)),
            pl.BlockSpec((d, 2), lambda g, i: (0, 0)),
            pl.BlockSpec((1, 2), lambda g, i: (0, 0)),
        ],
        out_specs=[
            pl.BlockSpec((1, BN, d), lambda g, i: (g, i, 0)),
            pl.BlockSpec((1, BN, 1), lambda g, i: (g, i, 0)),
            pl.BlockSpec((1, BN, 1), lambda g, i: (g, i, 0)),
            pl.BlockSpec((1, BN, 2), lambda g, i: (g, i, 0)),
        ],
        out_shape=[
            jax.ShapeDtypeStruct((G, n, d), jnp.float32),
            jax.ShapeDtypeStruct((G, n, 1), jnp.float32),
            jax.ShapeDtypeStruct((G, n, 1), jnp.float32),
            jax.ShapeDtypeStruct((G, n, 2), jnp.float32),
        ],
    )(PW, PD, Hm, EXS, b2, Ws, bs, Wy1, by1, Wy0, by0, Wp, bp)


def _make_ex_kernel(n):
    mesh = plsc.VectorSubcoreMesh(core_axis_name="c", subcore_axis_name="s")

    @functools.partial(
        pl.kernel,
        mesh=mesh,
        compiler_params=pltpu.CompilerParams(needs_layout_passes=False),
        out_type=[
            jax.ShapeDtypeStruct((2 * EPAD,), jnp.float32),
            jax.ShapeDtypeStruct((2 * NW * NPAD,), jnp.float32),
        ],
        scratch_types=[
            pltpu.VMEM((CBATCH, 1, CHUNK), jnp.int32),
            pltpu.VMEM((CBATCH, 1, CHUNK), jnp.int32),
            pltpu.VMEM((NPAD,), jnp.float32),
            pltpu.VMEM((NPAD,), jnp.float32),
            pltpu.VMEM((NPAD,), jnp.float32),
            pltpu.VMEM((CBATCH * CHUNK,), jnp.float32),
        ],
    )
    def ex_kernel(als_hbm, ald_hbm, src_hbm, dst_hbm, ex_hbm, outd_hbm,
                  srcv, dstv, alsv, aldv, denv, exstage):
        cid = lax.axis_index("c")
        sid = lax.axis_index("s")
        wid = cid * NS + sid
        zero16 = jnp.zeros((LANES,), jnp.float32)

        for g in range(2):
            def zeroden(i, carry):
                denv[pl.ds(i * LANES, LANES)] = zero16
                return carry
            lax.fori_loop(0, NPAD // LANES, zeroden, 0)

            pltpu.sync_copy(als_hbm.at[pl.ds(g * NPAD, NPAD)], alsv)
            pltpu.sync_copy(ald_hbm.at[pl.ds(g * NPAD, NPAD)], aldv)

            def batch_body(b, carry):
                pltpu.sync_copy(src_hbm.at[g, wid, pl.ds(b * CBATCH, CBATCH)],
                                srcv)
                pltpu.sync_copy(dst_hbm.at[g, wid, pl.ds(b * CBATCH, CBATCH)],
                                dstv)

                def chunk_body(c, carry2):
                    for k in range(CHUNK // LANES):
                        s16 = srcv[c, 0, pl.ds(k * LANES, LANES)]
                        d16 = dstv[c, 0, pl.ds(k * LANES, LANES)]
                        a = (plsc.load_gather(alsv, [s16])
                             + plsc.load_gather(aldv, [d16]))
                        ex16 = jnp.exp(_leaky(a, 0.2))
                        plsc.addupdate_scatter(denv, [d16], ex16)
                        exstage[pl.ds(c * CHUNK + k * LANES, LANES)] = ex16
                    return carry2
                lax.fori_loop(0, CBATCH, chunk_body, 0)

                off = ((g * NW + wid) * NCHUNKS + b * CBATCH) * CHUNK
                pltpu.sync_copy(exstage, ex_hbm.at[pl.ds(off, CBATCH * CHUNK)])
                return carry
            lax.fori_loop(0, NCHUNKS // CBATCH, batch_body, 0)

            pltpu.sync_copy(denv,
                            outd_hbm.at[pl.ds((g * NW + wid) * NPAD, NPAD)])

    return ex_kernel


def _make_row_kernel(n, d):
    mesh = plsc.VectorSubcoreMesh(core_axis_name="c", subcore_axis_name="s")
    rows_per_tec = NPAD // NS
    npairs = CBATCH // 2

    @functools.partial(
        pl.kernel,
        mesh=mesh,
        compiler_params=pltpu.CompilerParams(needs_layout_passes=False),
        out_type=jax.ShapeDtypeStruct((2, NC, NPAD, d), jnp.float32),
        scratch_types=[
            pltpu.VMEM_SHARED((NPAD, d), jnp.float32),
            pltpu.VMEM((CBATCH, 1, CHUNK), jnp.int32),
            pltpu.VMEM((CBATCH, 1, CHUNK), jnp.int32),
            pltpu.VMEM((CBATCH * CHUNK,), jnp.float32),
            pltpu.VMEM((CHUNK, d), jnp.float32),
            pltpu.VMEM((CHUNK, d), jnp.float32),
            pltpu.VMEM((CHUNK, d), jnp.float32),
            pltpu.VMEM((CHUNK, d), jnp.float32),
            pltpu.SemaphoreType.DMA,
            pltpu.SemaphoreType.DMA,
            pltpu.SemaphoreType.DMA,
            pltpu.SemaphoreType.DMA,
        ],
    )
    def row_kernel(h0_hbm, h1_hbm, src_hbm, dst_hbm, ex_hbm, outw_hbm,
                   acc, srcv, dstv, exv, gb0, gb1, sb0, sb1,
                   gsem0, gsem1, ssem0, ssem1):
        cid = lax.axis_index("c")
        sid = lax.axis_index("s")
        wid = cid * NS + sid
        r0 = sid * rows_per_tec
        zero16 = jnp.zeros((LANES,), jnp.float32)
        gbufs = (gb0, gb1)
        sbufs = (sb0, sb1)
        gsems = (gsem0, gsem1)
        ssems = (ssem0, ssem1)

        for g in range(2):
            h_hbm = h0_hbm if g == 0 else h1_hbm

            def zerorows(i, carry):
                for k in range(d // LANES):
                    gb0[i, pl.ds(k * LANES, LANES)] = zero16
                return carry
            lax.fori_loop(0, CHUNK, zerorows, 0)

            def zeroacc(i, carry):
                pltpu.sync_copy(gb0, acc.at[pl.ds(r0 + i * CHUNK, CHUNK)])
                return carry
            lax.fori_loop(0, rows_per_tec // CHUNK, zeroacc, 0)
            plsc.subcore_barrier()

            def batch_body(b, carry):
                pltpu.sync_copy(src_hbm.at[g, wid, pl.ds(b * CBATCH, CBATCH)],
                                srcv)
                pltpu.sync_copy(dst_hbm.at[g, wid, pl.ds(b * CBATCH, CBATCH)],
                                dstv)
                off = ((g * NW + wid) * NCHUNKS + b * CBATCH) * CHUNK
                pltpu.sync_copy(ex_hbm.at[pl.ds(off, CBATCH * CHUNK)], exv)

                for bb in range(2):
                    pltpu.async_copy(h_hbm.at[srcv.at[bb, 0]], gbufs[bb],
                                     gsems[bb])

                def pair_body(q, carry2):
                    for bb in range(2):
                        cix = q * 2 + bb
                        gb, sb = gbufs[bb], sbufs[bb]
                        pltpu.make_async_copy(
                            h_hbm.at[srcv.at[cix, 0]], gb, gsems[bb]).wait()

                        @pl.when(q > 0)
                        def _():
                            pltpu.make_async_copy(
                                sb, acc.at[dstv.at[cix, 0]],
                                ssems[bb]).wait()

                        for k in range(CHUNK // LANES):
                            ex16 = exv[pl.ds(cix * CHUNK + k * LANES, LANES)]
                            for j in range(LANES):
                                e = ex16[j]
                                row = k * LANES + j
                                for kk in range(d // LANES):
                                    sl = pl.ds(kk * LANES, LANES)
                                    sb[row, sl] = gb[row, sl] * e

                        pltpu.async_copy(sb, acc.at[dstv.at[cix, 0]],
                                         ssems[bb], add=True)

                        @pl.when(q < npairs - 1)
                        def _():
                            pltpu.async_copy(
                                h_hbm.at[srcv.at[cix + 2, 0]], gb, gsems[bb])
                    return carry2
                lax.fori_loop(0, npairs, pair_body, 0)

                for bb in range(2):
                    pltpu.make_async_copy(sbufs[bb], acc.at[dstv.at[bb, 0]],
                                          ssems[bb]).wait()
                return carry
            lax.fori_loop(0, NCHUNKS // CBATCH, batch_body, 0)
            plsc.subcore_barrier()

            pltpu.sync_copy(acc.at[pl.ds(r0, rows_per_tec)],
                            outw_hbm.at[g, cid, pl.ds(r0, rows_per_tec)])
            plsc.subcore_barrier()

    return row_kernel


def _make_head_gather(n):
    mesh = plsc.VectorSubcoreMesh(core_axis_name="c", subcore_axis_name="s")
    per_w = TPAD // NW

    @functools.partial(
        pl.kernel,
        mesh=mesh,
        compiler_params=pltpu.CompilerParams(needs_layout_passes=False),
        out_type=jax.ShapeDtypeStruct((4 * TPAD,), jnp.float32),
        scratch_types=[
            pltpu.VMEM((n,), jnp.float32),
            pltpu.VMEM((n,), jnp.float32),
            pltpu.VMEM((n,), jnp.float32),
            pltpu.VMEM((n,), jnp.float32),
            pltpu.VMEM((per_w,), jnp.int32),
            pltpu.VMEM((per_w,), jnp.int32),
            pltpu.VMEM((per_w,), jnp.float32),
            pltpu.VMEM((per_w,), jnp.float32),
            pltpu.VMEM((per_w,), jnp.float32),
            pltpu.VMEM((per_w,), jnp.float32),
        ],
    )
    def head_kernel(u1_hbm, u0_hbm, ti_hbm, ci_hbm, y_hbm,
                    u1r, u1f, u0r, u0f, tiv, civ, yb0, yb1, yb2, yb3):
        cid = lax.axis_index("c")
        sid = lax.axis_index("s")
        wid = cid * NS + sid
        r0 = wid * per_w
        pltpu.sync_copy(u1_hbm.at[pl.ds(0, n)], u1r)
        pltpu.sync_copy(u1_hbm.at[pl.ds(n, n)], u1f)
        pltpu.sync_copy(u0_hbm.at[pl.ds(0, n)], u0r)
        pltpu.sync_copy(u0_hbm.at[pl.ds(n, n)], u0f)
        pltpu.sync_copy(ti_hbm.at[pl.ds(r0, per_w)], tiv)
        pltpu.sync_copy(ci_hbm.at[pl.ds(r0, per_w)], civ)
        for i in range(per_w // LANES):
            sl = pl.ds(i * LANES, LANES)
            t16 = tiv[sl]
            c16 = civ[sl]
            yb0[sl] = plsc.load_gather(u1r, [t16])
            yb1[sl] = plsc.load_gather(u0f, [t16])
            yb2[sl] = plsc.load_gather(u0r, [c16])
            yb3[sl] = plsc.load_gather(u1f, [c16])
        for k, yb in enumerate((yb0, yb1, yb2, yb3)):
            pltpu.sync_copy(yb, y_hbm.at[pl.ds(k * TPAD + r0, per_w)])

    return head_kernel


def kernel(x, edge_index, fake_x, fake_edge_index, treat_idx, control_idx,
           W1, as1, ad1, b1, W2, as2, ad2, b2, Ws, bs, Wy1, by1, Wy0, by0,
           Wp, bp):
    n, d = x.shape
    e = edge_index.shape[1]
    t = treat_idx.shape[0]

    pad = EPAD - e
    padsrc = jnp.zeros((pad,), jnp.int32)
    paddst = n + (jnp.arange(pad, dtype=jnp.int32) % (NPAD - n))
    SRC = jnp.stack([
        jnp.concatenate([edge_index[0], padsrc]),
        jnp.concatenate([fake_edge_index[0], padsrc]),
    ]).reshape(2, NW, NCHUNKS, 1, CHUNK)
    DST = jnp.stack([
        jnp.concatenate([edge_index[1], paddst]),
        jnp.concatenate([fake_edge_index[1], paddst]),
    ]).reshape(2, NW, NCHUNKS, 1, CHUNK)

    X = jnp.stack([x, fake_x])
    ex_kernel = _make_ex_kernel(n)
    row_kernel = _make_row_kernel(n, d)

    def _padal(A):
        return jnp.pad(A.reshape(2, n), ((0, 0), (0, NPAD - n))).reshape(-1)

    H1, ALS1, ALD1, EXS1 = _dense_attn(X, W1, as1.reshape(1, d),
                                       ad1.reshape(1, d))
    EX1, PD1 = ex_kernel(_padal(ALS1), _padal(ALD1), SRC, DST)
    PW1 = row_kernel(H1[0], H1[1], SRC, DST, EX1)
    Z1 = _combine1(PW1, PD1.reshape(2, NW, NPAD, 1),
                   H1, EXS1, b1.reshape(1, d))

    H2, ALS2, ALD2, EXS2 = _dense_attn(Z1, W2, as2.reshape(1, d),
                                       ad2.reshape(1, d))
    EX2, PD2 = ex_kernel(_padal(ALS2), _padal(ALD2), SRC, DST)
    PW2 = row_kernel(H2[0], H2[1], SRC, DST, EX2)
    Z2, U1, U0, TP = _combine2_heads(
        PW2, PD2.reshape(2, NW, NPAD, 1),
        H2, EXS2, b2.reshape(1, d),
        Ws, bs.reshape(1, d), Wy1, by1.reshape(1, 1), Wy0, by0.reshape(1, 1),
        Wp, bp.reshape(1, 2))

    tpad = TPAD - t
    ti = jnp.concatenate([treat_idx, jnp.zeros((tpad,), jnp.int32)])
    ci = jnp.concatenate([control_idx, jnp.zeros((tpad,), jnp.int32)])
    head_kernel = _make_head_gather(n)
    Y = head_kernel(U1.reshape(2 * n), U0.reshape(2 * n), ti, ci).reshape(4, TPAD)

    return (Y[0, :t], Y[1, :t], Y[2, :t], Y[3, :t], Z2[0], Z2[1], TP[0])
